# baseline (speedup 1.0000x reference)
import functools

import jax
import jax.numpy as jnp
from jax import lax
from jax.experimental import pallas as pl
from jax.experimental.pallas import tpu as pltpu
from jax.experimental.pallas import tpu_sc as plsc


def _sc_geometry():
    try:
        info = plsc.get_sparse_core_info()
        return info.num_cores, info.num_subcores
    except Exception:
        return 2, 16


def _pick_chunk(E, NW):
    for ch in range(128, 0, -1):
        if E % (NW * ch) == 0 and (E // (NW * ch)) % 8 == 0:
            return ch, E // (NW * ch)
    return None, None


def _make_degree_kernel(NPAD, CH, RPW, NC, NS):
    mesh = plsc.VectorSubcoreMesh(core_axis_name="c", subcore_axis_name="s")
    rps = NPAD // NS
    out_sds = jax.ShapeDtypeStruct((NPAD,), jnp.float32)

    @functools.partial(
        pl.kernel,
        out_type=(out_sds, out_sds),
        mesh=mesh,
        scratch_types=[
            pltpu.VMEM((RPW, CH), jnp.int32),
            pltpu.VMEM((-(-CH // 16) * 16,), jnp.float32),
            pltpu.VMEM((rps,), jnp.float32),
            pltpu.VMEM_SHARED((NPAD,), jnp.float32),
            pltpu.SemaphoreType.DMA,
        ],
        compiler_params=pltpu.CompilerParams(use_tc_tiling_on_sc=False),
    )
    def deg_kernel(edge_hbm, out0_hbm, out1_hbm, idx_v, ones_v, zbuf, acc_sh,
                   sem):
        c = lax.axis_index("c")
        s = lax.axis_index("s")
        wid = c * NS + s

        def zloop(i, carry):
            zbuf[pl.ds(i * 16, 16)] = jnp.zeros((16,), jnp.float32)
            return carry

        lax.fori_loop(0, rps // 16, zloop, 0)
        for t in range(-(-CH // 16)):
            ones_v[pl.ds(t * 16, 16)] = jnp.ones((16,), jnp.float32)
        pltpu.sync_copy(zbuf, acc_sh.at[pl.ds(s * rps, rps)])
        plsc.subcore_barrier()

        pltpu.sync_copy(edge_hbm.at[1, pl.ds(wid * RPW, RPW)], idx_v)

        def body(j, carry):
            pltpu.sync_copy(ones_v.at[pl.ds(0, CH)], acc_sh.at[idx_v.at[j]],
                            add=True)
            return carry

        lax.fori_loop(0, RPW, body, 0)
        plsc.subcore_barrier()

        @pl.when(c == 0)
        def _():
            pltpu.sync_copy(acc_sh.at[pl.ds(s * rps, rps)],
                            out0_hbm.at[pl.ds(s * rps, rps)])

        @pl.when(c == 1)
        def _():
            pltpu.sync_copy(acc_sh.at[pl.ds(s * rps, rps)],
                            out1_hbm.at[pl.ds(s * rps, rps)])

    return deg_kernel


def _make_scatter_kernel(N, NPAD, F, CH, RPW, NC, NS, K=10):
    mesh = plsc.VectorSubcoreMesh(core_axis_name="c", subcore_axis_name="s")
    rps = NPAD // NS
    assert RPW % K == 0
    M = RPW // K
    out_sds = jax.ShapeDtypeStruct((NPAD, F), jnp.float32)

    @functools.partial(
        pl.kernel,
        out_type=(out_sds, out_sds),
        mesh=mesh,
        scratch_types=[
            pltpu.VMEM((RPW, CH), jnp.int32),
            pltpu.VMEM((RPW, CH), jnp.int32),
            pltpu.VMEM((K, CH, F), jnp.float32),
            pltpu.VMEM((rps, F), jnp.float32),
            pltpu.VMEM_SHARED((NPAD, F), jnp.float32),
            pltpu.SemaphoreType.DMA((K,)),
            pltpu.SemaphoreType.DMA((K,)),
        ],
        compiler_params=pltpu.CompilerParams(use_tc_tiling_on_sc=False),
    )
    def scat_kernel(g_hbm, edge_hbm, out0_hbm, out1_hbm,
                    sidx, didx, rows_v, zbuf, acc_sh, gsem, ssem):
        c = lax.axis_index("c")
        s = lax.axis_index("s")
        wid = c * NS + s

        def zloop(i, carry):
            for t in range(F // 16):
                zbuf[i, pl.ds(t * 16, 16)] = jnp.zeros((16,), jnp.float32)
            return carry

        lax.fori_loop(0, rps, zloop, 0)
        pltpu.sync_copy(zbuf, acc_sh.at[pl.ds(s * rps, rps)])
        plsc.subcore_barrier()

        pltpu.sync_copy(edge_hbm.at[0, pl.ds(wid * RPW, RPW)], sidx)
        pltpu.sync_copy(edge_hbm.at[1, pl.ds(wid * RPW, RPW)], didx)

        def gather(j, b):
            pltpu.async_copy(g_hbm.at[sidx.at[j]], rows_v.at[b], gsem.at[b])

        def wait_gather(j, b):
            pltpu.make_async_copy(g_hbm.at[sidx.at[j]], rows_v.at[b],
                                  gsem.at[b]).wait()

        def scatter(j, b):
            pltpu.async_copy(rows_v.at[b], acc_sh.at[didx.at[j]], ssem.at[b],
                             add=True)

        def wait_scatter(j, b):
            pltpu.make_async_copy(g_hbm.at[sidx.at[j]], rows_v.at[b],
                                  ssem.at[b]).wait()

        for b in range(K):
            gather(b, b)

        def body(m, carry):
            for b in range(K):
                j = m * K + b
                wait_gather(j, b)
                scatter(j, b)
            for b in range(K):
                j = m * K + b
                wait_scatter(j, b)
                gather(j + K, b)
            return carry

        lax.fori_loop(0, M - 1, body, 0)
        for b in range(K):
            j = (M - 1) * K + b
            wait_gather(j, b)
            scatter(j, b)
        for b in range(K):
            j = (M - 1) * K + b
            wait_scatter(j, b)

        plsc.subcore_barrier()

        @pl.when(c == 0)
        def _():
            pltpu.sync_copy(acc_sh.at[pl.ds(s * rps, rps)],
                            out0_hbm.at[pl.ds(s * rps, rps)])

        @pl.when(c == 1)
        def _():
            pltpu.sync_copy(acc_sh.at[pl.ds(s * rps, rps)],
                            out1_hbm.at[pl.ds(s * rps, rps)])

    return scat_kernel


def _tc_matmul(x, W):
    N, D = x.shape
    F = W.shape[1]

    def body(x_ref, w_ref, o_ref):
        o_ref[...] = jnp.dot(x_ref[...], w_ref[...],
                             preferred_element_type=jnp.float32)

    return pl.pallas_call(
        body,
        out_shape=jax.ShapeDtypeStruct((N, F), jnp.float32),
    )(x, W)


def _tc_scale(h1, d0, d1):
    N, F = h1.shape

    def body(h_ref, d0_ref, d1_ref, g_ref, dinv_ref):
        deg = d0_ref[...] + d1_ref[...] + 1.0
        dinv = lax.rsqrt(deg)[:N, None]
        dinv_ref[...] = dinv
        g_ref[...] = h_ref[...] * dinv

    return pl.pallas_call(
        body,
        out_shape=(
            jax.ShapeDtypeStruct((N, F), jnp.float32),
            jax.ShapeDtypeStruct((N, 1), jnp.float32),
        ),
    )(h1, d0, d1)


def _tc_mid(a0, a1, g1, dinv, b1, W2):
    N, F = g1.shape
    F2 = W2.shape[1]

    def body(a0_ref, a1_ref, g_ref, dinv_ref, b_ref, w_ref, o_ref):
        dinv = dinv_ref[...]
        z = dinv * (a0_ref[:N] + a1_ref[:N] + g_ref[...]) + b_ref[...]
        z = jnp.maximum(z, 0.0)
        h = jnp.dot(z, w_ref[...], preferred_element_type=jnp.float32)
        o_ref[...] = h * dinv

    return pl.pallas_call(
        body,
        out_shape=jax.ShapeDtypeStruct((N, F2), jnp.float32),
    )(a0, a1, g1, dinv, b1, W2)


def _tc_final(a0, a1, g2, dinv, b2, Wfc, bfc):
    N, F = g2.shape

    def body(a0_ref, a1_ref, g_ref, dinv_ref, b_ref, w_ref, bf_ref, o_ref):
        dinv = dinv_ref[...]
        z = dinv * (a0_ref[:N] + a1_ref[:N] + g_ref[...]) + b_ref[...]
        z = jnp.maximum(z, 0.0)
        o = jnp.dot(z, w_ref[...],
                    preferred_element_type=jnp.float32) + bf_ref[...]
        o_ref[...] = o[:, 0]

    return pl.pallas_call(
        body,
        out_shape=jax.ShapeDtypeStruct((N,), jnp.float32),
    )(a0, a1, g2, dinv, b2, Wfc, bfc)


def kernel(x, edge_index, W1, b1, W2, b2, Wfc, bfc):
    N, D = x.shape
    E = edge_index.shape[1]
    F1 = W1.shape[1]
    F2 = W2.shape[1]
    NC, NS = _sc_geometry()
    NW = NC * NS

    CH, RPW = _pick_chunk(E, NW)
    NPAD = -(-N // 512) * 512
    if CH is None:
        CH = 128
        RPW = -(-(-(-E // (NW * CH))) // 8) * 8
        EPAD = RPW * NW * CH
        if NPAD == N:
            NPAD += 512
        pidx = jnp.arange(EPAD - E, dtype=edge_index.dtype)
        pad = jnp.stack([pidx % N, N + pidx % (NPAD - N)])
        edge_index = jnp.concatenate([edge_index, pad], axis=1)
        E = EPAD
    edge3d = edge_index.reshape(2, E // CH, CH)

    h1 = _tc_matmul(x, W1)
    d0, d1 = _make_degree_kernel(NPAD, CH, RPW, NC, NS)(edge3d)

    g1, dinv = _tc_scale(h1, d0, d1)
    a10, a11 = _make_scatter_kernel(N, NPAD, F1, CH, RPW, NC, NS)(g1, edge3d)
    g2 = _tc_mid(a10, a11, g1, dinv, b1.reshape(1, F1), W2)
    a20, a21 = _make_scatter_kernel(N, NPAD, F2, CH, RPW, NC, NS)(g2, edge3d)
    return _tc_final(a20, a21, g2, dinv, b2.reshape(1, F2), Wfc,
                     bfc.reshape(1, 1))


# speedup vs baseline: 1.0085x; 1.0085x over previous
import functools

import jax
import jax.numpy as jnp
from jax import lax
from jax.experimental import pallas as pl
from jax.experimental.pallas import tpu as pltpu
from jax.experimental.pallas import tpu_sc as plsc


def _sc_geometry():
    try:
        info = plsc.get_sparse_core_info()
        return info.num_cores, info.num_subcores
    except Exception:
        return 2, 16


def _pick_chunk(E, NW):
    for ch in range(128, 0, -1):
        if E % (NW * ch) == 0 and (E // (NW * ch)) % 8 == 0:
            return ch, E // (NW * ch)
    return None, None


def _make_degree_kernel(NPAD, CH, RPW, NC, NS):
    mesh = plsc.VectorSubcoreMesh(core_axis_name="c", subcore_axis_name="s")
    rps = NPAD // NS
    out_sds = jax.ShapeDtypeStruct((NPAD,), jnp.float32)

    @functools.partial(
        pl.kernel,
        out_type=(out_sds, out_sds),
        mesh=mesh,
        scratch_types=[
            pltpu.VMEM((RPW, CH), jnp.int32),
            pltpu.VMEM((-(-CH // 16) * 16,), jnp.float32),
            pltpu.VMEM((rps,), jnp.float32),
            pltpu.VMEM_SHARED((NPAD,), jnp.float32),
            pltpu.SemaphoreType.DMA,
        ],
        compiler_params=pltpu.CompilerParams(use_tc_tiling_on_sc=False),
    )
    def deg_kernel(edge_hbm, out0_hbm, out1_hbm, idx_v, ones_v, zbuf, acc_sh,
                   sem):
        c = lax.axis_index("c")
        s = lax.axis_index("s")
        wid = c * NS + s

        def zloop(i, carry):
            zbuf[pl.ds(i * 16, 16)] = jnp.zeros((16,), jnp.float32)
            return carry

        lax.fori_loop(0, rps // 16, zloop, 0)
        for t in range(-(-CH // 16)):
            ones_v[pl.ds(t * 16, 16)] = jnp.ones((16,), jnp.float32)
        pltpu.sync_copy(zbuf, acc_sh.at[pl.ds(s * rps, rps)])
        plsc.subcore_barrier()

        pltpu.sync_copy(edge_hbm.at[1, pl.ds(wid * RPW, RPW)], idx_v)

        def body(j, carry):
            pltpu.sync_copy(ones_v.at[pl.ds(0, CH)], acc_sh.at[idx_v.at[j]],
                            add=True)
            return carry

        lax.fori_loop(0, RPW, body, 0)
        plsc.subcore_barrier()

        @pl.when(c == 0)
        def _():
            pltpu.sync_copy(acc_sh.at[pl.ds(s * rps, rps)],
                            out0_hbm.at[pl.ds(s * rps, rps)])

        @pl.when(c == 1)
        def _():
            pltpu.sync_copy(acc_sh.at[pl.ds(s * rps, rps)],
                            out1_hbm.at[pl.ds(s * rps, rps)])

    return deg_kernel


def _make_scatter_kernel(N, NPAD, F, CH, RPW, NC, NS, K=8):
    mesh = plsc.VectorSubcoreMesh(core_axis_name="c", subcore_axis_name="s")
    rps = NPAD // NS
    assert RPW % K == 0
    M = RPW // K
    out_sds = jax.ShapeDtypeStruct((NPAD, F), jnp.float32)

    @functools.partial(
        pl.kernel,
        out_type=(out_sds, out_sds),
        mesh=mesh,
        scratch_types=[
            pltpu.VMEM((RPW, CH), jnp.int32),
            pltpu.VMEM((RPW, CH), jnp.int32),
            pltpu.VMEM((K, CH, F), jnp.float32),
            pltpu.VMEM((rps, F), jnp.float32),
            pltpu.VMEM_SHARED((NPAD, F), jnp.float32),
            pltpu.SemaphoreType.DMA((K,)),
            pltpu.SemaphoreType.DMA((K,)),
        ],
        compiler_params=pltpu.CompilerParams(use_tc_tiling_on_sc=False),
    )
    def scat_kernel(g_hbm, edge_hbm, out0_hbm, out1_hbm,
                    sidx, didx, rows_v, zbuf, acc_sh, gsem, ssem):
        c = lax.axis_index("c")
        s = lax.axis_index("s")
        wid = c * NS + s

        def zloop(i, carry):
            for t in range(F // 16):
                zbuf[i, pl.ds(t * 16, 16)] = jnp.zeros((16,), jnp.float32)
            return carry

        lax.fori_loop(0, rps, zloop, 0)
        pltpu.sync_copy(zbuf, acc_sh.at[pl.ds(s * rps, rps)])
        plsc.subcore_barrier()

        pltpu.sync_copy(edge_hbm.at[0, pl.ds(wid * RPW, RPW)], sidx)
        pltpu.sync_copy(edge_hbm.at[1, pl.ds(wid * RPW, RPW)], didx)

        def gather(j, b):
            pltpu.async_copy(g_hbm.at[sidx.at[j]], rows_v.at[b], gsem.at[b])

        def wait_gather(j, b):
            pltpu.make_async_copy(g_hbm.at[sidx.at[j]], rows_v.at[b],
                                  gsem.at[b]).wait()

        def scatter(j, b):
            pltpu.async_copy(rows_v.at[b], acc_sh.at[didx.at[j]], ssem.at[b],
                             add=True)

        def wait_scatter(j, b):
            pltpu.make_async_copy(g_hbm.at[sidx.at[j]], rows_v.at[b],
                                  ssem.at[b]).wait()

        for b in range(K):
            gather(b, b)

        def body(m, carry):
            for b in range(K):
                j = m * K + b
                wait_gather(j, b)
                scatter(j, b)
            for b in range(K):
                j = m * K + b
                wait_scatter(j, b)
                gather(j + K, b)
            return carry

        lax.fori_loop(0, M - 1, body, 0)
        for b in range(K):
            j = (M - 1) * K + b
            wait_gather(j, b)
            scatter(j, b)
        for b in range(K):
            j = (M - 1) * K + b
            wait_scatter(j, b)

        plsc.subcore_barrier()

        @pl.when(c == 0)
        def _():
            pltpu.sync_copy(acc_sh.at[pl.ds(s * rps, rps)],
                            out0_hbm.at[pl.ds(s * rps, rps)])

        @pl.when(c == 1)
        def _():
            pltpu.sync_copy(acc_sh.at[pl.ds(s * rps, rps)],
                            out1_hbm.at[pl.ds(s * rps, rps)])

    return scat_kernel


def _tc_matmul(x, W):
    N, D = x.shape
    F = W.shape[1]

    def body(x_ref, w_ref, o_ref):
        o_ref[...] = jnp.dot(x_ref[...], w_ref[...],
                             preferred_element_type=jnp.float32)

    return pl.pallas_call(
        body,
        out_shape=jax.ShapeDtypeStruct((N, F), jnp.float32),
    )(x, W)


def _tc_scale(h1, d0, d1):
    N, F = h1.shape

    def body(h_ref, d0_ref, d1_ref, g_ref, dinv_ref):
        deg = d0_ref[...] + d1_ref[...] + 1.0
        dinv = lax.rsqrt(deg)[:N, None]
        dinv_ref[...] = dinv
        g_ref[...] = h_ref[...] * dinv

    return pl.pallas_call(
        body,
        out_shape=(
            jax.ShapeDtypeStruct((N, F), jnp.float32),
            jax.ShapeDtypeStruct((N, 1), jnp.float32),
        ),
    )(h1, d0, d1)


def _tc_mid(a0, a1, g1, dinv, b1, W2):
    N, F = g1.shape
    F2 = W2.shape[1]

    def body(a0_ref, a1_ref, g_ref, dinv_ref, b_ref, w_ref, o_ref):
        dinv = dinv_ref[...]
        z = dinv * (a0_ref[:N] + a1_ref[:N] + g_ref[...]) + b_ref[...]
        z = jnp.maximum(z, 0.0)
        h = jnp.dot(z, w_ref[...], preferred_element_type=jnp.float32)
        o_ref[...] = h * dinv

    return pl.pallas_call(
        body,
        out_shape=jax.ShapeDtypeStruct((N, F2), jnp.float32),
    )(a0, a1, g1, dinv, b1, W2)


def _tc_final(a0, a1, g2, dinv, b2, Wfc, bfc):
    N, F = g2.shape

    def body(a0_ref, a1_ref, g_ref, dinv_ref, b_ref, w_ref, bf_ref, o_ref):
        dinv = dinv_ref[...]
        z = dinv * (a0_ref[:N] + a1_ref[:N] + g_ref[...]) + b_ref[...]
        z = jnp.maximum(z, 0.0)
        o = jnp.dot(z, w_ref[...],
                    preferred_element_type=jnp.float32) + bf_ref[...]
        o_ref[...] = o[:, 0]

    return pl.pallas_call(
        body,
        out_shape=jax.ShapeDtypeStruct((N,), jnp.float32),
    )(a0, a1, g2, dinv, b2, Wfc, bfc)


def kernel(x, edge_index, W1, b1, W2, b2, Wfc, bfc):
    N, D = x.shape
    E = edge_index.shape[1]
    F1 = W1.shape[1]
    F2 = W2.shape[1]
    NC, NS = _sc_geometry()
    NW = NC * NS

    CH, RPW = _pick_chunk(E, NW)
    NPAD = -(-N // 512) * 512
    if CH is None:
        CH = 128
        RPW = -(-(-(-E // (NW * CH))) // 8) * 8
        EPAD = RPW * NW * CH
        if NPAD == N:
            NPAD += 512
        pidx = jnp.arange(EPAD - E, dtype=edge_index.dtype)
        pad = jnp.stack([pidx % N, N + pidx % (NPAD - N)])
        edge_index = jnp.concatenate([edge_index, pad], axis=1)
        E = EPAD
    edge3d = edge_index.reshape(2, E // CH, CH)

    h1 = _tc_matmul(x, W1)
    d0, d1 = _make_degree_kernel(NPAD, CH, RPW, NC, NS)(edge3d)

    g1, dinv = _tc_scale(h1, d0, d1)
    a10, a11 = _make_scatter_kernel(N, NPAD, F1, CH, RPW, NC, NS)(g1, edge3d)
    g2 = _tc_mid(a10, a11, g1, dinv, b1.reshape(1, F1), W2)
    a20, a21 = _make_scatter_kernel(N, NPAD, F2, CH, RPW, NC, NS)(g2, edge3d)
    return _tc_final(a20, a21, g2, dinv, b2.reshape(1, F2), Wfc,
                     bfc.reshape(1, 1))
